# baseline (speedup 1.0000x reference)
import functools

import jax
import jax.numpy as jnp
from jax import lax
from jax.experimental import pallas as pl
from jax.experimental.pallas import tpu as pltpu
from jax.experimental.pallas import tpu_sc as plsc

_NC = 2
_NS = 16
_NW = _NC * _NS
_CHUNK = 128
_BR = 10240


def _sc_mesh():
    return plsc.VectorSubcoreMesh(core_axis_name="c", subcore_axis_name="s")


def _deg_kernel(n_pad, n_chunks):
    @functools.partial(
        pl.kernel,
        mesh=_sc_mesh(),
        out_type=jax.ShapeDtypeStruct((_NW, 2, n_pad), jnp.float32),
        compiler_params=pltpu.CompilerParams(needs_layout_passes=False),
        scratch_types=[
            pltpu.VMEM((n_chunks, _CHUNK), jnp.int32),
            pltpu.VMEM((n_chunks, _CHUNK), jnp.int32),
            pltpu.VMEM((n_pad,), jnp.float32),
            pltpu.VMEM((n_pad,), jnp.float32),
            pltpu.SemaphoreType.DMA,
        ],
    )
    def deg(src_hbm, dst_hbm, out_hbm, sidx, didx, hist_o, hist_i, isem):
        c = lax.axis_index("c")
        s = lax.axis_index("s")
        w = s * _NC + c
        ones16 = jnp.ones((16,), jnp.float32)

        pltpu.async_copy(src_hbm.at[w], sidx, isem)
        pltpu.async_copy(dst_hbm.at[w], didx, isem)

        def fill_zeros(i, carry):
            hist_o[pl.ds(i * 16, 16)] = jnp.zeros((16,), jnp.float32)
            hist_i[pl.ds(i * 16, 16)] = jnp.zeros((16,), jnp.float32)
            return carry

        lax.fori_loop(0, n_pad // 16, fill_zeros, 0)
        pltpu.make_async_copy(src_hbm.at[w], sidx, isem).wait()
        pltpu.make_async_copy(dst_hbm.at[w], didx, isem).wait()

        def step(j, carry):
            for i in range(_CHUNK // 16):
                iv_s = sidx[j, pl.ds(i * 16, 16)]
                iv_d = didx[j, pl.ds(i * 16, 16)]
                plsc.addupdate_scatter(hist_o, [iv_s], ones16)
                plsc.addupdate_scatter(hist_i, [iv_d], ones16)
            return carry

        lax.fori_loop(0, n_chunks, step, 0)
        pltpu.sync_copy(hist_o, out_hbm.at[w, 0])
        pltpu.sync_copy(hist_i, out_hbm.at[w, 1])

    return deg


def _agg_kernel(n_pad, n_chunks, d):
    rpt = n_pad // _NS
    nz = rpt // _CHUNK

    @functools.partial(
        pl.kernel,
        mesh=_sc_mesh(),
        out_type=jax.ShapeDtypeStruct((_NC, n_pad, d), jnp.float32),
        scratch_types=[
            pltpu.VMEM((3, _CHUNK), jnp.int32),
            pltpu.VMEM((3, _CHUNK), jnp.int32),
            pltpu.VMEM((nz, _CHUNK), jnp.int32),
            pltpu.VMEM((2, _CHUNK, d), jnp.float32),
            pltpu.VMEM_SHARED((n_pad, d), jnp.float32),
            pltpu.SemaphoreType.DMA,
            pltpu.SemaphoreType.DMA,
            pltpu.SemaphoreType.DMA,
        ],
    )
    def agg(src_hbm, dst_hbm, hs_hbm, out_hbm, sidx, didx, ridx, rows, acc,
            gsem, ssem, isem):
        c = lax.axis_index("c")
        s = lax.axis_index("s")
        w = s * _NC + c
        base = s * rpt

        pltpu.async_copy(src_hbm.at[w, 0], sidx.at[0], isem)
        pltpu.async_copy(dst_hbm.at[w, 0], didx.at[0], isem)

        def zrow(i, carry):
            for k in range(d // 16):
                rows[0, i, pl.ds(k * 16, 16)] = jnp.zeros((16,), jnp.float32)
            return carry

        lax.fori_loop(0, _CHUNK, zrow, 0)
        for k in range(nz):
            for i in range(_CHUNK // 16):
                ridx[k, pl.ds(i * 16, 16)] = (
                    base + k * _CHUNK + i * 16 + lax.iota(jnp.int32, 16))
        for k in range(nz):
            pltpu.async_copy(rows.at[0], acc.at[ridx.at[k]], ssem)
        for k in range(nz):
            pltpu.make_async_copy(rows.at[0], acc.at[ridx.at[k]], ssem).wait()
        plsc.subcore_barrier()

        pltpu.make_async_copy(src_hbm.at[w, 0], sidx.at[0], isem).wait()
        pltpu.make_async_copy(dst_hbm.at[w, 0], didx.at[0], isem).wait()
        if n_chunks > 1:
            pltpu.async_copy(src_hbm.at[w, 1], sidx.at[1], isem)
            pltpu.async_copy(dst_hbm.at[w, 1], didx.at[1], isem)
        pltpu.async_copy(hs_hbm.at[sidx.at[0]], rows.at[0], gsem)

        def step(j, carry):
            b = lax.rem(j, 2)
            q = lax.rem(j, 3)
            qn = lax.rem(j + 1, 3)
            qp = lax.rem(j + 2, 3)
            pltpu.make_async_copy(hs_hbm.at[sidx.at[q]], rows.at[b],
                                  gsem).wait()

            @pl.when(j + 1 < n_chunks)
            def _next_gather():
                pltpu.make_async_copy(src_hbm.at[w, j], sidx.at[qn],
                                      isem).wait()
                pltpu.make_async_copy(dst_hbm.at[w, j], didx.at[qn],
                                      isem).wait()

                @pl.when(j >= 1)
                def _wait_prev_scatter():
                    pltpu.make_async_copy(rows.at[1 - b], acc.at[ridx.at[0]],
                                          ssem).wait()

                pltpu.async_copy(hs_hbm.at[sidx.at[qn]], rows.at[1 - b],
                                 gsem)

            @pl.when(j + 2 < n_chunks)
            def _prefetch_idx():
                pltpu.async_copy(src_hbm.at[w, j + 2], sidx.at[qp], isem)
                pltpu.async_copy(dst_hbm.at[w, j + 2], didx.at[qp], isem)

            pltpu.async_copy(rows.at[b], acc.at[didx.at[q]], ssem, add=True)
            return carry

        lax.fori_loop(0, n_chunks, step, 0)
        b_last = lax.rem(n_chunks - 1, 2)
        if n_chunks > 1:
            pltpu.make_async_copy(rows.at[1 - b_last], acc.at[ridx.at[0]],
                                  ssem).wait()
        pltpu.make_async_copy(rows.at[b_last], acc.at[ridx.at[0]],
                              ssem).wait()
        plsc.subcore_barrier()
        pltpu.async_copy(acc.at[ridx.at[0]], rows.at[0], gsem)
        for k in range(nz):
            bb = k % 2
            pltpu.make_async_copy(acc.at[ridx.at[k]], rows.at[bb],
                                  gsem).wait()
            if k + 1 < nz:
                if k >= 1:
                    pltpu.make_async_copy(
                        rows.at[1 - bb],
                        out_hbm.at[c, pl.ds(base + (k - 1) * _CHUNK, _CHUNK)],
                        ssem).wait()
                pltpu.async_copy(acc.at[ridx.at[k + 1]], rows.at[1 - bb],
                                 gsem)
            pltpu.async_copy(rows.at[bb],
                             out_hbm.at[c, pl.ds(base + k * _CHUNK, _CHUNK)],
                             ssem)
        for k in range(max(nz - 2, 0), nz):
            bb = k % 2
            pltpu.make_async_copy(rows.at[bb],
                                  out_hbm.at[c, pl.ds(base + k * _CHUNK,
                                                      _CHUNK)], ssem).wait()

    return agg


def _dinv(deg_ref):
    deg = jnp.sum(deg_ref[...], axis=1, keepdims=True)
    return lax.rsqrt(jnp.maximum(deg, 1.0))


def _dot(a, w):
    return jnp.dot(a, w, preferred_element_type=jnp.float32,
                   precision=lax.Precision.HIGHEST)


def _embed_body(x_ref, w_ref, b_ref, o_ref):
    o_ref[...] = _dot(x_ref[...], w_ref[...]) + b_ref[...]


def _scale_body(h_ref, dgo_ref, o_ref):
    o_ref[...] = h_ref[...] * _dinv(dgo_ref)


def _embed_scale_body(x_ref, dgo_ref, w_ref, b_ref, o_ref):
    o_ref[...] = (_dot(x_ref[...], w_ref[...]) + b_ref[...]) * _dinv(dgo_ref)


def _mid_body(a0_ref, a1_ref, dgi_ref, dgo_ref, w_ref, b_ref, o_ref):
    agg = (a0_ref[...] + a1_ref[...]) * _dinv(dgi_ref)
    o_ref[...] = (_dot(agg, w_ref[...]) + b_ref[...]) * _dinv(dgo_ref)


def _head_body(a0_ref, a1_ref, dgi_ref, w1_ref, b1_ref, w2_ref, b2_ref, o_ref):
    agg = (a0_ref[...] + a1_ref[...]) * _dinv(dgi_ref)
    h = _dot(agg, w1_ref[...]) + b1_ref[...]
    o_ref[...] = _dot(h, w2_ref[...]) + b2_ref[...]


def _row_call(body, n_pad, d, in_specs):
    return pl.pallas_call(
        body,
        grid=(n_pad // _BR,),
        in_specs=in_specs,
        out_specs=pl.BlockSpec((_BR, d), lambda i: (i, 0)),
        out_shape=jax.ShapeDtypeStruct((n_pad, d), jnp.float32),
    )


def _rows_spec(d):
    return pl.BlockSpec((_BR, d), lambda i: (i, 0))


def _deg_spec():
    return pl.BlockSpec((_BR, _NW), lambda i: (i, 0))


def _w_spec(d):
    return pl.BlockSpec((d, d), lambda i: (0, 0))


def _b_spec(d):
    return pl.BlockSpec((1, d), lambda i: (0, 0))


def kernel(x, edge_index, W_embed, b_embed, W_g0, b_g0, W_g1, b_g1, W_dec, b_dec):
    n, d = x.shape
    e = edge_index.shape[1]
    n_pad = -(-(n + 1) // 2048) * 2048
    n_chunks = -(-e // (_NW * _CHUNK))
    e_pad = n_chunks * _NW * _CHUNK

    src = edge_index[0].astype(jnp.int32)
    dst = edge_index[1].astype(jnp.int32)
    pad_idx = n + (jnp.arange(e_pad - e, dtype=jnp.int32) % (n_pad - n))
    src_p = jnp.concatenate([src, pad_idx]).reshape(_NW, n_chunks, _CHUNK)
    dst_p = jnp.concatenate([dst, pad_idx]).reshape(_NW, n_chunks, _CHUNK)
    x_pad = jnp.pad(x, ((0, n_pad - n), (0, 0)))
    b_embed2 = b_embed.reshape(1, d)
    b_g02 = b_g0.reshape(1, d)
    b_g12 = b_g1.reshape(1, d)
    b_dec2 = b_dec.reshape(1, d)

    degp = _deg_kernel(n_pad, n_chunks)(src_p, dst_p)
    dgo = degp[:, 0].T
    dgi = degp[:, 1].T

    agg = _agg_kernel(n_pad, n_chunks, d)

    hs1 = _row_call(_embed_scale_body, n_pad, d,
                    [_rows_spec(d), _deg_spec(), _w_spec(d), _b_spec(d)])(
                        x_pad, dgo, W_embed, b_embed2)
    p1 = agg(src_p, dst_p, hs1)
    hs2 = _row_call(_mid_body, n_pad, d,
                    [_rows_spec(d), _rows_spec(d), _deg_spec(), _deg_spec(),
                     _w_spec(d), _b_spec(d)])(
                        p1[0], p1[1], dgi, dgo, W_g0, b_g02)
    p2 = agg(src_p, dst_p, hs2)
    out = _row_call(_head_body, n_pad, d,
                    [_rows_spec(d), _rows_spec(d), _deg_spec(),
                     _w_spec(d), _b_spec(d), _w_spec(d), _b_spec(d)])(
                        p2[0], p2[1], dgi, W_g1, b_g12, W_dec, b_dec2)
    return out[:n]


# speedup vs baseline: 1.0184x; 1.0184x over previous
import functools

import jax
import jax.numpy as jnp
from jax import lax
from jax.experimental import pallas as pl
from jax.experimental.pallas import tpu as pltpu
from jax.experimental.pallas import tpu_sc as plsc

_NC = 2
_NS = 16
_NW = _NC * _NS
_CHUNK = 128
_BR = 2048


def _sc_mesh():
    return plsc.VectorSubcoreMesh(core_axis_name="c", subcore_axis_name="s")


def _deg_kernel(n_pad, n_chunks):
    @functools.partial(
        pl.kernel,
        mesh=_sc_mesh(),
        out_type=jax.ShapeDtypeStruct((_NW, 2, n_pad), jnp.float32),
        compiler_params=pltpu.CompilerParams(needs_layout_passes=False),
        scratch_types=[
            pltpu.VMEM((n_chunks, _CHUNK), jnp.int32),
            pltpu.VMEM((n_chunks, _CHUNK), jnp.int32),
            pltpu.VMEM((n_pad,), jnp.float32),
            pltpu.VMEM((n_pad,), jnp.float32),
            pltpu.SemaphoreType.DMA,
        ],
    )
    def deg(src_hbm, dst_hbm, out_hbm, sidx, didx, hist_o, hist_i, isem):
        c = lax.axis_index("c")
        s = lax.axis_index("s")
        w = s * _NC + c
        ones16 = jnp.ones((16,), jnp.float32)

        pltpu.async_copy(src_hbm.at[w], sidx, isem)
        pltpu.async_copy(dst_hbm.at[w], didx, isem)

        def fill_zeros(i, carry):
            hist_o[pl.ds(i * 16, 16)] = jnp.zeros((16,), jnp.float32)
            hist_i[pl.ds(i * 16, 16)] = jnp.zeros((16,), jnp.float32)
            return carry

        lax.fori_loop(0, n_pad // 16, fill_zeros, 0)
        pltpu.make_async_copy(src_hbm.at[w], sidx, isem).wait()
        pltpu.make_async_copy(dst_hbm.at[w], didx, isem).wait()

        def step(j, carry):
            for i in range(_CHUNK // 16):
                iv_s = sidx[j, pl.ds(i * 16, 16)]
                iv_d = didx[j, pl.ds(i * 16, 16)]
                plsc.addupdate_scatter(hist_o, [iv_s], ones16)
                plsc.addupdate_scatter(hist_i, [iv_d], ones16)
            return carry

        lax.fori_loop(0, n_chunks, step, 0)
        pltpu.sync_copy(hist_o, out_hbm.at[w, 0])
        pltpu.sync_copy(hist_i, out_hbm.at[w, 1])

    return deg


def _agg_kernel(n_pad, n_chunks, d):
    rpt = n_pad // _NS
    nz = rpt // _CHUNK

    @functools.partial(
        pl.kernel,
        mesh=_sc_mesh(),
        out_type=jax.ShapeDtypeStruct((_NC, n_pad, d), jnp.float32),
        scratch_types=[
            pltpu.VMEM((3, _CHUNK), jnp.int32),
            pltpu.VMEM((3, _CHUNK), jnp.int32),
            pltpu.VMEM((nz, _CHUNK), jnp.int32),
            pltpu.VMEM((2, _CHUNK, d), jnp.float32),
            pltpu.VMEM_SHARED((n_pad, d), jnp.float32),
            pltpu.SemaphoreType.DMA,
            pltpu.SemaphoreType.DMA,
            pltpu.SemaphoreType.DMA,
        ],
    )
    def agg(src_hbm, dst_hbm, hs_hbm, out_hbm, sidx, didx, ridx, rows, acc,
            gsem, ssem, isem):
        c = lax.axis_index("c")
        s = lax.axis_index("s")
        w = s * _NC + c
        base = s * rpt

        pltpu.async_copy(src_hbm.at[w, 0], sidx.at[0], isem)
        pltpu.async_copy(dst_hbm.at[w, 0], didx.at[0], isem)

        def zrow(i, carry):
            for k in range(d // 16):
                rows[0, i, pl.ds(k * 16, 16)] = jnp.zeros((16,), jnp.float32)
            return carry

        lax.fori_loop(0, _CHUNK, zrow, 0)
        for k in range(nz):
            for i in range(_CHUNK // 16):
                ridx[k, pl.ds(i * 16, 16)] = (
                    base + k * _CHUNK + i * 16 + lax.iota(jnp.int32, 16))
        for k in range(nz):
            pltpu.async_copy(rows.at[0], acc.at[ridx.at[k]], ssem)
        for k in range(nz):
            pltpu.make_async_copy(rows.at[0], acc.at[ridx.at[k]], ssem).wait()
        plsc.subcore_barrier()

        pltpu.make_async_copy(src_hbm.at[w, 0], sidx.at[0], isem).wait()
        pltpu.make_async_copy(dst_hbm.at[w, 0], didx.at[0], isem).wait()
        if n_chunks > 1:
            pltpu.async_copy(src_hbm.at[w, 1], sidx.at[1], isem)
            pltpu.async_copy(dst_hbm.at[w, 1], didx.at[1], isem)
        pltpu.async_copy(hs_hbm.at[sidx.at[0]], rows.at[0], gsem)

        def step(j, carry):
            b = lax.rem(j, 2)
            q = lax.rem(j, 3)
            qn = lax.rem(j + 1, 3)
            qp = lax.rem(j + 2, 3)
            pltpu.make_async_copy(hs_hbm.at[sidx.at[q]], rows.at[b],
                                  gsem).wait()

            @pl.when(j + 1 < n_chunks)
            def _next_gather():
                pltpu.make_async_copy(src_hbm.at[w, j], sidx.at[qn],
                                      isem).wait()
                pltpu.make_async_copy(dst_hbm.at[w, j], didx.at[qn],
                                      isem).wait()

                @pl.when(j >= 1)
                def _wait_prev_scatter():
                    pltpu.make_async_copy(rows.at[1 - b], acc.at[ridx.at[0]],
                                          ssem).wait()

                pltpu.async_copy(hs_hbm.at[sidx.at[qn]], rows.at[1 - b],
                                 gsem)

            @pl.when(j + 2 < n_chunks)
            def _prefetch_idx():
                pltpu.async_copy(src_hbm.at[w, j + 2], sidx.at[qp], isem)
                pltpu.async_copy(dst_hbm.at[w, j + 2], didx.at[qp], isem)

            pltpu.async_copy(rows.at[b], acc.at[didx.at[q]], ssem, add=True)
            return carry

        lax.fori_loop(0, n_chunks, step, 0)
        b_last = lax.rem(n_chunks - 1, 2)
        if n_chunks > 1:
            pltpu.make_async_copy(rows.at[1 - b_last], acc.at[ridx.at[0]],
                                  ssem).wait()
        pltpu.make_async_copy(rows.at[b_last], acc.at[ridx.at[0]],
                              ssem).wait()
        plsc.subcore_barrier()
        pltpu.async_copy(acc.at[ridx.at[0]], rows.at[0], gsem)
        for k in range(nz):
            bb = k % 2
            pltpu.make_async_copy(acc.at[ridx.at[k]], rows.at[bb],
                                  gsem).wait()
            if k + 1 < nz:
                if k >= 1:
                    pltpu.make_async_copy(
                        rows.at[1 - bb],
                        out_hbm.at[c, pl.ds(base + (k - 1) * _CHUNK, _CHUNK)],
                        ssem).wait()
                pltpu.async_copy(acc.at[ridx.at[k + 1]], rows.at[1 - bb],
                                 gsem)
            pltpu.async_copy(rows.at[bb],
                             out_hbm.at[c, pl.ds(base + k * _CHUNK, _CHUNK)],
                             ssem)
        for k in range(max(nz - 2, 0), nz):
            bb = k % 2
            pltpu.make_async_copy(rows.at[bb],
                                  out_hbm.at[c, pl.ds(base + k * _CHUNK,
                                                      _CHUNK)], ssem).wait()

    return agg


def _dinv(deg_ref):
    deg = jnp.sum(deg_ref[...], axis=1, keepdims=True)
    return lax.rsqrt(jnp.maximum(deg, 1.0))


def _dot(a, w):
    return jnp.dot(a, w, preferred_element_type=jnp.float32,
                   precision=lax.Precision.HIGHEST)


def _embed_body(x_ref, w_ref, b_ref, o_ref):
    o_ref[...] = _dot(x_ref[...], w_ref[...]) + b_ref[...]


def _scale_body(h_ref, dgo_ref, o_ref):
    o_ref[...] = h_ref[...] * _dinv(dgo_ref)


def _embed_scale_body(x_ref, dgo_ref, w_ref, b_ref, o_ref):
    o_ref[...] = (_dot(x_ref[...], w_ref[...]) + b_ref[...]) * _dinv(dgo_ref)


def _mid_body(a0_ref, a1_ref, dgi_ref, dgo_ref, w_ref, b_ref, o_ref):
    agg = (a0_ref[...] + a1_ref[...]) * _dinv(dgi_ref)
    o_ref[...] = (_dot(agg, w_ref[...]) + b_ref[...]) * _dinv(dgo_ref)


def _head_body(a0_ref, a1_ref, dgi_ref, w1_ref, b1_ref, w2_ref, b2_ref, o_ref):
    agg = (a0_ref[...] + a1_ref[...]) * _dinv(dgi_ref)
    h = _dot(agg, w1_ref[...]) + b1_ref[...]
    o_ref[...] = _dot(h, w2_ref[...]) + b2_ref[...]


def _row_call(body, n_pad, d, in_specs):
    return pl.pallas_call(
        body,
        grid=(n_pad // _BR,),
        in_specs=in_specs,
        out_specs=pl.BlockSpec((_BR, d), lambda i: (i, 0)),
        out_shape=jax.ShapeDtypeStruct((n_pad, d), jnp.float32),
    )


def _rows_spec(d):
    return pl.BlockSpec((_BR, d), lambda i: (i, 0))


def _deg_spec():
    return pl.BlockSpec((_BR, _NW), lambda i: (i, 0))


def _w_spec(d):
    return pl.BlockSpec((d, d), lambda i: (0, 0))


def _b_spec(d):
    return pl.BlockSpec((1, d), lambda i: (0, 0))


def kernel(x, edge_index, W_embed, b_embed, W_g0, b_g0, W_g1, b_g1, W_dec, b_dec):
    n, d = x.shape
    e = edge_index.shape[1]
    n_pad = -(-(n + 1) // 2048) * 2048
    n_chunks = -(-e // (_NW * _CHUNK))
    e_pad = n_chunks * _NW * _CHUNK

    src = edge_index[0].astype(jnp.int32)
    dst = edge_index[1].astype(jnp.int32)
    pad_idx = n + (jnp.arange(e_pad - e, dtype=jnp.int32) % (n_pad - n))
    src_p = jnp.concatenate([src, pad_idx]).reshape(_NW, n_chunks, _CHUNK)
    dst_p = jnp.concatenate([dst, pad_idx]).reshape(_NW, n_chunks, _CHUNK)
    x_pad = jnp.pad(x, ((0, n_pad - n), (0, 0)))
    b_embed2 = b_embed.reshape(1, d)
    b_g02 = b_g0.reshape(1, d)
    b_g12 = b_g1.reshape(1, d)
    b_dec2 = b_dec.reshape(1, d)

    degp = _deg_kernel(n_pad, n_chunks)(src_p, dst_p)
    dgo = degp[:, 0].T
    dgi = degp[:, 1].T

    agg = _agg_kernel(n_pad, n_chunks, d)

    hs1 = _row_call(_embed_scale_body, n_pad, d,
                    [_rows_spec(d), _deg_spec(), _w_spec(d), _b_spec(d)])(
                        x_pad, dgo, W_embed, b_embed2)
    p1 = agg(src_p, dst_p, hs1)
    hs2 = _row_call(_mid_body, n_pad, d,
                    [_rows_spec(d), _rows_spec(d), _deg_spec(), _deg_spec(),
                     _w_spec(d), _b_spec(d)])(
                        p1[0], p1[1], dgi, dgo, W_g0, b_g02)
    p2 = agg(src_p, dst_p, hs2)
    out = _row_call(_head_body, n_pad, d,
                    [_rows_spec(d), _rows_spec(d), _deg_spec(),
                     _w_spec(d), _b_spec(d), _w_spec(d), _b_spec(d)])(
                        p2[0], p2[1], dgi, W_g1, b_g12, W_dec, b_dec2)
    return out[:n]


# speedup vs baseline: 1.0216x; 1.0032x over previous
import functools

import jax
import jax.numpy as jnp
from jax import lax
from jax.experimental import pallas as pl
from jax.experimental.pallas import tpu as pltpu
from jax.experimental.pallas import tpu_sc as plsc

_NC = 2
_NS = 16
_NW = _NC * _NS
_CHUNK = 128
_BR = 2048


def _sc_mesh():
    return plsc.VectorSubcoreMesh(core_axis_name="c", subcore_axis_name="s")


def _deg_kernel(n_pad, n_chunks):
    @functools.partial(
        pl.kernel,
        mesh=_sc_mesh(),
        out_type=jax.ShapeDtypeStruct((_NW, 2, n_pad), jnp.float32),
        compiler_params=pltpu.CompilerParams(needs_layout_passes=False),
        scratch_types=[
            pltpu.VMEM((n_chunks, _CHUNK), jnp.int32),
            pltpu.VMEM((n_chunks, _CHUNK), jnp.int32),
            pltpu.VMEM((n_pad,), jnp.float32),
            pltpu.VMEM((n_pad,), jnp.float32),
            pltpu.SemaphoreType.DMA,
        ],
    )
    def deg(src_hbm, dst_hbm, out_hbm, sidx, didx, hist_o, hist_i, isem):
        c = lax.axis_index("c")
        s = lax.axis_index("s")
        w = s * _NC + c
        ones16 = jnp.ones((16,), jnp.float32)

        pltpu.async_copy(src_hbm.at[w], sidx, isem)
        pltpu.async_copy(dst_hbm.at[w], didx, isem)

        def fill_zeros(i, carry):
            hist_o[pl.ds(i * 16, 16)] = jnp.zeros((16,), jnp.float32)
            hist_i[pl.ds(i * 16, 16)] = jnp.zeros((16,), jnp.float32)
            return carry

        lax.fori_loop(0, n_pad // 16, fill_zeros, 0)
        pltpu.make_async_copy(src_hbm.at[w], sidx, isem).wait()
        pltpu.make_async_copy(dst_hbm.at[w], didx, isem).wait()

        def step(j, carry):
            for i in range(_CHUNK // 16):
                iv_s = sidx[j, pl.ds(i * 16, 16)]
                iv_d = didx[j, pl.ds(i * 16, 16)]
                plsc.addupdate_scatter(hist_o, [iv_s], ones16)
                plsc.addupdate_scatter(hist_i, [iv_d], ones16)
            return carry

        lax.fori_loop(0, n_chunks, step, 0)
        pltpu.sync_copy(hist_o, out_hbm.at[w, 0])
        pltpu.sync_copy(hist_i, out_hbm.at[w, 1])

    return deg


def _agg_kernel(n_pad, n_chunks, d):
    rpt = n_pad // _NS
    nz = rpt // _CHUNK

    @functools.partial(
        pl.kernel,
        mesh=_sc_mesh(),
        out_type=jax.ShapeDtypeStruct((_NC, n_pad, d), jnp.float32),
        scratch_types=[
            pltpu.VMEM((3, _CHUNK), jnp.int32),
            pltpu.VMEM((3, _CHUNK), jnp.int32),
            pltpu.VMEM((nz, _CHUNK), jnp.int32),
            pltpu.VMEM((2, _CHUNK, d), jnp.float32),
            pltpu.VMEM_SHARED((n_pad, d), jnp.float32),
            pltpu.SemaphoreType.DMA,
            pltpu.SemaphoreType.DMA,
            pltpu.SemaphoreType.DMA,
        ],
    )
    def agg(src_hbm, dst_hbm, hs_hbm, out_hbm, sidx, didx, ridx, rows, acc,
            gsem, ssem, isem):
        c = lax.axis_index("c")
        s = lax.axis_index("s")
        w = s * _NC + c
        base = s * rpt

        pltpu.async_copy(src_hbm.at[w, 0], sidx.at[0], isem)
        pltpu.async_copy(dst_hbm.at[w, 0], didx.at[0], isem)

        def zrow(i, carry):
            for k in range(d // 16):
                rows[0, i, pl.ds(k * 16, 16)] = jnp.zeros((16,), jnp.float32)
            return carry

        lax.fori_loop(0, _CHUNK, zrow, 0)
        for k in range(nz):
            for i in range(_CHUNK // 16):
                ridx[k, pl.ds(i * 16, 16)] = (
                    base + k * _CHUNK + i * 16 + lax.iota(jnp.int32, 16))
        for k in range(nz):
            pltpu.async_copy(rows.at[0], acc.at[ridx.at[k]], ssem)
        for k in range(nz):
            pltpu.make_async_copy(rows.at[0], acc.at[ridx.at[k]], ssem).wait()
        plsc.subcore_barrier()

        pltpu.make_async_copy(src_hbm.at[w, 0], sidx.at[0], isem).wait()
        pltpu.make_async_copy(dst_hbm.at[w, 0], didx.at[0], isem).wait()
        if n_chunks > 1:
            pltpu.async_copy(src_hbm.at[w, 1], sidx.at[1], isem)
            pltpu.async_copy(dst_hbm.at[w, 1], didx.at[1], isem)
        pltpu.async_copy(hs_hbm.at[sidx.at[0]], rows.at[0], gsem)

        def step(j, carry):
            b = lax.rem(j, 2)
            q = lax.rem(j, 3)
            qn = lax.rem(j + 1, 3)
            qp = lax.rem(j + 2, 3)
            pltpu.make_async_copy(hs_hbm.at[sidx.at[q]], rows.at[b],
                                  gsem).wait()

            @pl.when(j + 1 < n_chunks)
            def _next_gather():
                pltpu.make_async_copy(src_hbm.at[w, j], sidx.at[qn],
                                      isem).wait()
                pltpu.make_async_copy(dst_hbm.at[w, j], didx.at[qn],
                                      isem).wait()

                @pl.when(j >= 1)
                def _wait_prev_scatter():
                    pltpu.make_async_copy(rows.at[1 - b], acc.at[ridx.at[0]],
                                          ssem).wait()

                pltpu.async_copy(hs_hbm.at[sidx.at[qn]], rows.at[1 - b],
                                 gsem)

            @pl.when(j + 2 < n_chunks)
            def _prefetch_idx():
                pltpu.async_copy(src_hbm.at[w, j + 2], sidx.at[qp], isem)
                pltpu.async_copy(dst_hbm.at[w, j + 2], didx.at[qp], isem)

            pltpu.async_copy(rows.at[b], acc.at[didx.at[q]], ssem, add=True)
            return carry

        lax.fori_loop(0, n_chunks, step, 0)
        b_last = lax.rem(n_chunks - 1, 2)
        if n_chunks > 1:
            pltpu.make_async_copy(rows.at[1 - b_last], acc.at[ridx.at[0]],
                                  ssem).wait()
        pltpu.make_async_copy(rows.at[b_last], acc.at[ridx.at[0]],
                              ssem).wait()
        plsc.subcore_barrier()
        pltpu.async_copy(acc.at[ridx.at[0]], rows.at[0], gsem)
        for k in range(nz):
            bb = k % 2
            pltpu.make_async_copy(acc.at[ridx.at[k]], rows.at[bb],
                                  gsem).wait()
            if k + 1 < nz:
                if k >= 1:
                    pltpu.make_async_copy(
                        rows.at[1 - bb],
                        out_hbm.at[c, pl.ds(base + (k - 1) * _CHUNK, _CHUNK)],
                        ssem).wait()
                pltpu.async_copy(acc.at[ridx.at[k + 1]], rows.at[1 - bb],
                                 gsem)
            pltpu.async_copy(rows.at[bb],
                             out_hbm.at[c, pl.ds(base + k * _CHUNK, _CHUNK)],
                             ssem)
        for k in range(max(nz - 2, 0), nz):
            bb = k % 2
            pltpu.make_async_copy(rows.at[bb],
                                  out_hbm.at[c, pl.ds(base + k * _CHUNK,
                                                      _CHUNK)], ssem).wait()

    return agg


def _dinv(deg_ref):
    deg = jnp.sum(deg_ref[...], axis=1, keepdims=True)
    return lax.rsqrt(jnp.maximum(deg, 1.0))


def _dot(a, w):
    return jnp.dot(a, w, preferred_element_type=jnp.float32,
                   precision=lax.Precision.HIGHEST)


def _embed_scale_body(x_ref, dgo_ref, w_ref, b_ref, o_ref):
    o_ref[...] = (_dot(x_ref[...], w_ref[...]) + b_ref[...]) * _dinv(dgo_ref)


def _mid_body(a0_ref, a1_ref, dgi_ref, dgo_ref, w_ref, b_ref, o_ref):
    agg = (a0_ref[...] + a1_ref[...]) * _dinv(dgi_ref)
    o_ref[...] = (_dot(agg, w_ref[...]) + b_ref[...]) * _dinv(dgo_ref)


def _head_body(a0_ref, a1_ref, dgi_ref, w1_ref, b1_ref, w2_ref, b2_ref, o_ref):
    agg = (a0_ref[...] + a1_ref[...]) * _dinv(dgi_ref)
    h = _dot(agg, w1_ref[...]) + b1_ref[...]
    o_ref[...] = _dot(h, w2_ref[...]) + b2_ref[...]


def _row_call(body, n_pad, d, in_specs):
    return pl.pallas_call(
        body,
        grid=(n_pad // _BR,),
        in_specs=in_specs,
        out_specs=pl.BlockSpec((_BR, d), lambda i: (i, 0)),
        out_shape=jax.ShapeDtypeStruct((n_pad, d), jnp.float32),
    )


def _rows_spec(d):
    return pl.BlockSpec((_BR, d), lambda i: (i, 0))


def _deg_spec():
    return pl.BlockSpec((_BR, _NW), lambda i: (i, 0))


def _w_spec(d):
    return pl.BlockSpec((d, d), lambda i: (0, 0))


def _b_spec(d):
    return pl.BlockSpec((1, d), lambda i: (0, 0))


def kernel(x, edge_index, W_embed, b_embed, W_g0, b_g0, W_g1, b_g1, W_dec, b_dec):
    n, d = x.shape
    e = edge_index.shape[1]
    n_pad = -(-(n + 1) // 2048) * 2048
    n_chunks = -(-e // (_NW * _CHUNK))
    e_pad = n_chunks * _NW * _CHUNK

    src = edge_index[0].astype(jnp.int32)
    dst = edge_index[1].astype(jnp.int32)
    pad_idx = n + (jnp.arange(e_pad - e, dtype=jnp.int32) % (n_pad - n))
    src_p = jnp.concatenate([src, pad_idx]).reshape(_NW, n_chunks, _CHUNK)
    dst_p = jnp.concatenate([dst, pad_idx]).reshape(_NW, n_chunks, _CHUNK)
    x_pad = jnp.pad(x, ((0, n_pad - n), (0, 0)))
    b_embed2 = b_embed.reshape(1, d)
    b_g02 = b_g0.reshape(1, d)
    b_g12 = b_g1.reshape(1, d)
    b_dec2 = b_dec.reshape(1, d)

    degp = _deg_kernel(n_pad, n_chunks)(src_p, dst_p)
    dgo = degp[:, 0].T
    dgi = degp[:, 1].T

    agg = _agg_kernel(n_pad, n_chunks, d)

    hs1 = _row_call(_embed_scale_body, n_pad, d,
                    [_rows_spec(d), _deg_spec(), _w_spec(d), _b_spec(d)])(
                        x_pad, dgo, W_embed, b_embed2)
    p1 = agg(src_p, dst_p, hs1)
    hs2 = _row_call(_mid_body, n_pad, d,
                    [_rows_spec(d), _rows_spec(d), _deg_spec(), _deg_spec(),
                     _w_spec(d), _b_spec(d)])(
                        p1[0], p1[1], dgi, dgo, W_g0, b_g02)
    p2 = agg(src_p, dst_p, hs2)
    out = _row_call(_head_body, n_pad, d,
                    [_rows_spec(d), _rows_spec(d), _deg_spec(),
                     _w_spec(d), _b_spec(d), _w_spec(d), _b_spec(d)])(
                        p2[0], p2[1], dgi, W_g1, b_g12, W_dec, b_dec2)
    return out[:n]
